# Initial kernel scaffold; baseline (speedup 1.0000x reference)
import jax
import jax.numpy as jnp
from jax.experimental import pallas as pl


def kernel(x, edge, W, a_l, a_r):
    raise NotImplementedError("write your pallas kernel here")


# speedup vs baseline: 25.2218x; 25.2218x over previous
import jax
import jax.numpy as jnp
from jax import lax
from jax.experimental import pallas as pl
from jax.experimental.pallas import tpu as pltpu
from jax.experimental.pallas import tpu_sc as plsc

N = 10000
F = 128
E = 320000
ALPHA = 0.2

NC = 2
NS = 16
NW = NC * NS
EPW = E // NW
K = 80
NCHUNK = EPW // K
NP = 10240
RPT = NP // NS
SPT = NP // NS


def _tc1_body(x_ref, w_ref, a2_ref, h_ref, al_ref, ar_ref):
    h = jnp.dot(x_ref[...], w_ref[...], preferred_element_type=jnp.float32)
    h_ref[...] = h
    alrt = lax.dot_general(
        a2_ref[...], h, (((1,), (1,)), ((), ())),
        preferred_element_type=jnp.float32)
    al_ref[...] = alrt[0]
    ar_ref[...] = alrt[1]


def _tc2_body(p_ref, s_ref, o_ref):
    ps = p_ref[0] + p_ref[1]
    ss = s_ref[0] + s_ref[1]
    o_ref[...] = ps / (ss + 1e-16)[:, None]


def _sc_body(h_hbm, row_hbm, col_hbm, al_hbm, ar_hbm, p_out, s_out,
             al_v, ar_v, row_v, col_v, w_v, rows_v, zs_v,
             acc_sh, s_sh, sem):
    c = lax.axis_index("c")
    sid = lax.axis_index("s")
    wid = sid * NC + c

    pltpu.sync_copy(al_hbm, al_v)
    pltpu.sync_copy(ar_hbm, ar_v)

    zero16 = jnp.zeros((16,), jnp.float32)

    def zrow_body(i, carry):
        for j in range(8):
            rows_v[i, pl.ds(j * 16, 16)] = zero16
        return carry

    lax.fori_loop(0, K, zrow_body, 0)
    for i in range(SPT // 16):
        zs_v[pl.ds(i * 16, 16)] = zero16
    for i in range(RPT // K):
        pltpu.sync_copy(rows_v, acc_sh.at[pl.ds(sid * RPT + i * K, K)])
    pltpu.sync_copy(zs_v, s_sh.at[pl.ds(sid * SPT, SPT)])
    plsc.subcore_barrier()

    e0 = wid * EPW

    def chunk_body(i, carry):
        off = e0 + i * K
        pltpu.sync_copy(row_hbm.at[pl.ds(off, K)], row_v)
        pltpu.sync_copy(col_hbm.at[pl.ds(off, K)], col_v)
        gat = pltpu.async_copy(h_hbm.at[col_v], rows_v, sem)
        for t in range(K // 16):
            r16 = row_v[pl.ds(t * 16, 16)]
            c16 = col_v[pl.ds(t * 16, 16)]
            a = plsc.load_gather(al_v, [r16])
            b = plsc.load_gather(ar_v, [c16])
            e = a + b
            e = jnp.where(e < 0.0, e * ALPHA, e)
            w_v[pl.ds(t * 16, 16)] = jnp.exp(e)
        gat.wait()
        for t in range(K // 16):
            w16 = w_v[pl.ds(t * 16, 16)]
            for l in range(16):
                ws = w16[l]
                ei = t * 16 + l
                for j in range(8):
                    sl = pl.ds(j * 16, 16)
                    rows_v[ei, sl] = rows_v[ei, sl] * ws
        pltpu.sync_copy(rows_v, acc_sh.at[row_v], add=True)
        pltpu.sync_copy(w_v, s_sh.at[row_v], add=True)
        return carry

    lax.fori_loop(0, NCHUNK, chunk_body, 0)
    plsc.subcore_barrier()

    for i in range(RPT // K):
        r0 = sid * RPT + i * K
        pltpu.sync_copy(acc_sh.at[pl.ds(r0, K)], rows_v)
        pltpu.sync_copy(rows_v, p_out.at[c, pl.ds(r0, K)])
    pltpu.sync_copy(s_sh.at[pl.ds(sid * SPT, SPT)], zs_v)
    pltpu.sync_copy(zs_v, s_out.at[c, pl.ds(sid * SPT, SPT)])


@jax.jit
def kernel(x, edge, W, a_l, a_r):
    a2 = jnp.concatenate(
        [a_l.reshape(1, F), a_r.reshape(1, F)], axis=0)
    row = edge[0].astype(jnp.int32)
    col = edge[1].astype(jnp.int32)

    B = 512
    grid = (N + B - 1) // B
    h, al, ar = pl.pallas_call(
        _tc1_body,
        grid=(grid,),
        in_specs=[
            pl.BlockSpec((B, F), lambda i: (i, 0)),
            pl.BlockSpec((F, F), lambda i: (0, 0)),
            pl.BlockSpec((2, F), lambda i: (0, 0)),
        ],
        out_specs=[
            pl.BlockSpec((B, F), lambda i: (i, 0)),
            pl.BlockSpec((B,), lambda i: (i,)),
            pl.BlockSpec((B,), lambda i: (i,)),
        ],
        out_shape=[
            jax.ShapeDtypeStruct((N, F), jnp.float32),
            jax.ShapeDtypeStruct((N,), jnp.float32),
            jax.ShapeDtypeStruct((N,), jnp.float32),
        ],
    )(x, W, a2)

    mesh = plsc.VectorSubcoreMesh(core_axis_name="c", subcore_axis_name="s")
    sc = pl.kernel(
        _sc_body,
        out_type=[
            jax.ShapeDtypeStruct((NC, NP, F), jnp.float32),
            jax.ShapeDtypeStruct((NC, NP), jnp.float32),
        ],
        mesh=mesh,
        compiler_params=pltpu.CompilerParams(needs_layout_passes=False),
        scratch_types=[
            pltpu.VMEM((N,), jnp.float32),
            pltpu.VMEM((N,), jnp.float32),
            pltpu.VMEM((K,), jnp.int32),
            pltpu.VMEM((K,), jnp.int32),
            pltpu.VMEM((K,), jnp.float32),
            pltpu.VMEM((K, F), jnp.float32),
            pltpu.VMEM((SPT,), jnp.float32),
            pltpu.VMEM_SHARED((NP, F), jnp.float32),
            pltpu.VMEM_SHARED((NP,), jnp.float32),
            pltpu.SemaphoreType.DMA,
        ],
    )
    p, s = sc(h, row, col, al, ar)

    out_pad = pl.pallas_call(
        _tc2_body,
        grid=(NP // B,),
        in_specs=[
            pl.BlockSpec((2, B, F), lambda i: (0, i, 0)),
            pl.BlockSpec((2, B), lambda i: (0, i)),
        ],
        out_specs=pl.BlockSpec((B, F), lambda i: (i, 0)),
        out_shape=jax.ShapeDtypeStruct((NP, F), jnp.float32),
    )(p, s)
    return out_pad[:N]


# speedup vs baseline: 43.4395x; 1.7223x over previous
import jax
import jax.numpy as jnp
from jax import lax
from jax.experimental import pallas as pl
from jax.experimental.pallas import tpu as pltpu
from jax.experimental.pallas import tpu_sc as plsc

N = 10000
F = 128
E = 320000
ALPHA = 0.2

NC = 2
NS = 16
NW = NC * NS
EPW = E // NW
K = 80
NCHUNK = EPW // K
NP = 10240
RPT = NP // NS
SPT = NP // NS


def _tc1_body(x_ref, w_ref, a2_ref, h_ref, al_ref, ar_ref):
    h = jnp.dot(x_ref[...], w_ref[...], preferred_element_type=jnp.float32)
    h_ref[...] = h
    alrt = lax.dot_general(
        a2_ref[...], h, (((1,), (1,)), ((), ())),
        preferred_element_type=jnp.float32)
    al_ref[...] = alrt[0]
    ar_ref[...] = alrt[1]


def _tc2_body(p_ref, s_ref, o_ref):
    ps = p_ref[0] + p_ref[1]
    ss = s_ref[0] + s_ref[1]
    o_ref[...] = ps / (ss + 1e-16)[:, None]


def _sc_body(h_hbm, row_hbm, col_hbm, al_hbm, ar_hbm, p_out, s_out,
             al_v, ar_v, row_v, col_v, w_v, rows_v, zs_v,
             acc_sh, s_sh, gsem0, gsem1, ssem0, ssem1, isem0, isem1):
    gsem = [gsem0, gsem1]
    ssem = [ssem0, ssem1]
    isem = [isem0, isem1]
    c = lax.axis_index("c")
    sid = lax.axis_index("s")
    wid = sid * NC + c

    pltpu.sync_copy(al_hbm, al_v)
    pltpu.sync_copy(ar_hbm, ar_v)

    zero16 = jnp.zeros((16,), jnp.float32)

    def zrow_body(i, carry):
        for j in range(8):
            rows_v[0, i, pl.ds(j * 16, 16)] = zero16
        return carry

    lax.fori_loop(0, K, zrow_body, 0)
    for i in range(SPT // 16):
        zs_v[pl.ds(i * 16, 16)] = zero16
    for i in range(RPT // K):
        pltpu.sync_copy(rows_v.at[0], acc_sh.at[pl.ds(sid * RPT + i * K, K)])
    pltpu.sync_copy(zs_v, s_sh.at[pl.ds(sid * SPT, SPT)])
    plsc.subcore_barrier()

    e0 = wid * EPW
    e_hi = e0 + EPW - K

    def idx_start(p, off):
        pltpu.async_copy(row_hbm.at[pl.ds(off, K)], row_v.at[p], isem[p])
        pltpu.async_copy(col_hbm.at[pl.ds(off, K)], col_v.at[p], isem[p])

    def idx_wait(p, off):
        pltpu.make_async_copy(
            row_hbm.at[pl.ds(off, K)], row_v.at[p], isem[p]).wait()
        pltpu.make_async_copy(
            col_hbm.at[pl.ds(off, K)], col_v.at[p], isem[p]).wait()

    def weights(p):
        for t in range(K // 16):
            r16 = row_v[p, pl.ds(t * 16, 16)]
            c16 = col_v[p, pl.ds(t * 16, 16)]
            a = plsc.load_gather(al_v, [r16])
            b = plsc.load_gather(ar_v, [c16])
            e = a + b
            e = jnp.where(e < 0.0, e * ALPHA, e)
            w_v[p, pl.ds(t * 16, 16)] = jnp.exp(e)

    def scale(p):
        for t in range(K // 16):
            w16 = w_v[p, pl.ds(t * 16, 16)]
            for l in range(16):
                ws = w16[l]
                ei = t * 16 + l
                for j in range(8):
                    sl = pl.ds(j * 16, 16)
                    rows_v[p, ei, sl] = rows_v[p, ei, sl] * ws

    def scatter_start(p):
        pltpu.async_copy(rows_v.at[p], acc_sh.at[row_v.at[p]], ssem[p],
                         add=True)
        pltpu.async_copy(w_v.at[p], s_sh.at[row_v.at[p]], ssem[p], add=True)

    def scatter_wait(p):
        pltpu.make_async_copy(
            rows_v.at[p], acc_sh.at[row_v.at[p]], ssem[p]).wait()
        pltpu.make_async_copy(
            w_v.at[p], s_sh.at[row_v.at[p]], ssem[p]).wait()

    def gather_start(p):
        pltpu.async_copy(h_hbm.at[col_v.at[p]], rows_v.at[p], gsem[p])

    def gather_wait(p):
        pltpu.make_async_copy(h_hbm.at[col_v.at[p]], rows_v.at[p],
                              gsem[p]).wait()

    idx_start(0, e0)
    idx_start(1, e0 + K)
    idx_wait(0, e0)
    gather_start(0)

    def pair_body(j, carry):
        a_off = e0 + 2 * j * K
        b_off = a_off + K
        weights(0)
        idx_wait(1, b_off)
        gather_start(1)
        gather_wait(0)
        scale(0)
        scatter_start(0)
        weights(1)
        gather_wait(1)
        scale(1)
        scatter_wait(0)
        idx_start(0, jnp.minimum(a_off + 2 * K, e_hi))
        scatter_start(1)
        scatter_wait(1)
        idx_start(1, jnp.minimum(b_off + 2 * K, e_hi))
        idx_wait(0, a_off)
        gather_start(0)
        return carry

    lax.fori_loop(0, (NCHUNK - 1) // 2, pair_body, 0)

    weights(0)
    pltpu.make_async_copy(h_hbm.at[col_v.at[0]], rows_v.at[0],
                          gsem[0]).wait()
    scale(0)
    scatter_start(0)
    scatter_wait(0)
    idx_wait(1, e_hi)
    plsc.subcore_barrier()

    for i in range(RPT // K):
        r0 = sid * RPT + i * K
        pltpu.sync_copy(acc_sh.at[pl.ds(r0, K)], rows_v.at[0])
        pltpu.sync_copy(rows_v.at[0], p_out.at[c, pl.ds(r0, K)])
    pltpu.sync_copy(s_sh.at[pl.ds(sid * SPT, SPT)], zs_v)
    pltpu.sync_copy(zs_v, s_out.at[c, pl.ds(sid * SPT, SPT)])


@jax.jit
def kernel(x, edge, W, a_l, a_r):
    a2 = jnp.concatenate(
        [a_l.reshape(1, F), a_r.reshape(1, F)], axis=0)
    row = edge[0].astype(jnp.int32)
    col = edge[1].astype(jnp.int32)

    B = 512
    grid = (N + B - 1) // B
    h, al, ar = pl.pallas_call(
        _tc1_body,
        grid=(grid,),
        in_specs=[
            pl.BlockSpec((B, F), lambda i: (i, 0)),
            pl.BlockSpec((F, F), lambda i: (0, 0)),
            pl.BlockSpec((2, F), lambda i: (0, 0)),
        ],
        out_specs=[
            pl.BlockSpec((B, F), lambda i: (i, 0)),
            pl.BlockSpec((B,), lambda i: (i,)),
            pl.BlockSpec((B,), lambda i: (i,)),
        ],
        out_shape=[
            jax.ShapeDtypeStruct((N, F), jnp.float32),
            jax.ShapeDtypeStruct((N,), jnp.float32),
            jax.ShapeDtypeStruct((N,), jnp.float32),
        ],
    )(x, W, a2)

    mesh = plsc.VectorSubcoreMesh(core_axis_name="c", subcore_axis_name="s")
    sc = pl.kernel(
        _sc_body,
        out_type=[
            jax.ShapeDtypeStruct((NC, NP, F), jnp.float32),
            jax.ShapeDtypeStruct((NC, NP), jnp.float32),
        ],
        mesh=mesh,
        compiler_params=pltpu.CompilerParams(needs_layout_passes=False),
        scratch_types=[
            pltpu.VMEM((N,), jnp.float32),
            pltpu.VMEM((N,), jnp.float32),
            pltpu.VMEM((2, K), jnp.int32),
            pltpu.VMEM((2, K), jnp.int32),
            pltpu.VMEM((2, K), jnp.float32),
            pltpu.VMEM((2, K, F), jnp.float32),
            pltpu.VMEM((SPT,), jnp.float32),
            pltpu.VMEM_SHARED((NP, F), jnp.float32),
            pltpu.VMEM_SHARED((NP,), jnp.float32),
            pltpu.SemaphoreType.DMA,
            pltpu.SemaphoreType.DMA,
            pltpu.SemaphoreType.DMA,
            pltpu.SemaphoreType.DMA,
            pltpu.SemaphoreType.DMA,
            pltpu.SemaphoreType.DMA,
        ],
    )
    p, s = sc(h, row, col, al, ar)

    out_pad = pl.pallas_call(
        _tc2_body,
        grid=(NP // B,),
        in_specs=[
            pl.BlockSpec((2, B, F), lambda i: (0, i, 0)),
            pl.BlockSpec((2, B), lambda i: (0, i)),
        ],
        out_specs=pl.BlockSpec((B, F), lambda i: (i, 0)),
        out_shape=jax.ShapeDtypeStruct((NP, F), jnp.float32),
    )(p, s)
    return out_pad[:N]
